# baseline (speedup 1.0000x reference)
import jax
import jax.numpy as jnp
from jax.experimental import pallas as pl
from jax.experimental.pallas import tpu as pltpu

CHAIN_LEN = 16
H = 128
N_CANDS = 64
MU = 0.5
K = 3 * H
NOUT = 5 * H


def _body(f_hbm, iou_hbm, c_hbm, sid_ref, siu_ref, vc_ref, bc_ref,
          wiou_ref, uiou_ref, biou_ref, wf_ref, wfb_ref, bf_ref, uf_ref,
          ufb_ref, wlt_ref, blin_ref,
          scores_ref, bv_ref, cat_ref, uw_ref, iou15_scr, c15_scr, acc_ref,
          sems):
    i = pl.program_id(0)
    nblocks = pl.num_programs(0)
    C = iou15_scr.shape[1]
    BC = bc_ref[...]

    def block_copies(blk, buf):
        cps = {}
        for t in range(CHAIN_LEN - 1, -1, -1):
            cps[t] = pltpu.make_async_copy(
                f_hbm.at[pl.ds(blk * C, C), t, :],
                cat_ref.at[buf, t, :, H:2 * H],
                sems.at[buf, t])
        cps['iou'] = pltpu.make_async_copy(
            iou_hbm.at[pl.ds(blk * C, C), CHAIN_LEN - 1, :],
            iou15_scr.at[buf], sems.at[buf, CHAIN_LEN])
        cps['c'] = pltpu.make_async_copy(
            c_hbm.at[pl.ds(blk * C, C), CHAIN_LEN - 1, :],
            c15_scr.at[buf], sems.at[buf, CHAIN_LEN + 1])
        return cps

    def start_all(cps):
        cps['iou'].start()
        cps['c'].start()
        for t in range(CHAIN_LEN - 1, -1, -1):
            cps[t].start()

    buf = jax.lax.rem(i, 2)
    nxt = jax.lax.rem(i + 1, 2)

    @pl.when(i == 0)
    def _init_const():
        lane = jax.lax.broadcasted_iota(jnp.int32, (C, H), 1)
        onehot = jnp.where(lane == 0, 1.0, 0.0)
        for b in range(2):
            for t in range(CHAIN_LEN):
                cat_ref[b, t, :, 2 * H:3 * H] = onehot
        acc_ref[...] = jnp.zeros_like(acc_ref)

        lane3 = jax.lax.broadcasted_iota(jnp.int32, (H, 3 * H), 1)
        sub3 = jax.lax.broadcasted_iota(jnp.int32, (H, 3 * H), 0)
        lane1 = jax.lax.broadcasted_iota(jnp.int32, (H, H), 1)
        sub1 = jax.lax.broadcasted_iota(jnp.int32, (H, H), 0)
        sc3 = jnp.where(lane3 < 2 * H, 0.5, 1.0)
        uw_ref[0:H, 0:3 * H] = jnp.swapaxes(uiou_ref[...], 0, 1) * sc3
        uw_ref[0:H, 3 * H:4 * H] = jnp.swapaxes(uf_ref[...], 0, 1) * 0.5
        uw_ref[0:H, 4 * H:5 * H] = jnp.where(
            lane1 == 0, jnp.broadcast_to(wlt_ref[...], (H, H)), 0.0)
        uw_ref[H:2 * H, 0:3 * H] = jnp.swapaxes(wiou_ref[...], 0, 1) * sc3
        uw_ref[H:2 * H, 3 * H:4 * H] = jnp.swapaxes(wf_ref[...], 0, 1) * 0.5
        uw_ref[H:2 * H, 4 * H:5 * H] = jnp.zeros((H, H), jnp.float32)
        biou_row = biou_ref[...] * jnp.where(
            lane3[0:1, :] < 2 * H, 0.5, 1.0)
        uw_ref[2 * H:3 * H, 0:3 * H] = jnp.where(
            sub3 == 0, jnp.broadcast_to(biou_row, (H, 3 * H)), 0.0)
        bias_f = (wfb_ref[...] + ufb_ref[...] + bf_ref[...]) * 0.5
        uw_ref[2 * H:3 * H, 3 * H:4 * H] = jnp.where(
            sub1 == 0, jnp.broadcast_to(bias_f, (H, H)), 0.0)
        uw_ref[2 * H:3 * H, 4 * H:5 * H] = jnp.where(
            (sub1 == 0) & (lane1 == 0), blin_ref[0, 0], 0.0)

    UW = uw_ref[...]

    @pl.when(i == 0)
    def _stage_first():
        start_all(block_copies(0, 0))

    @pl.when(i + 1 < nblocks)
    def _prefetch_next():
        start_all(block_copies(i + 1, nxt))

    waits = block_copies(i, buf)
    iou_cp = waits['iou']
    c_cp = waits['c']

    def gates(g3):
        ig = 0.5 * jnp.tanh(g3[:, :H]) + 0.5
        og = 0.5 * jnp.tanh(g3[:, H:2 * H]) + 0.5
        ug = jnp.tanh(g3[:, 2 * H:3 * H])
        return ig, og, ug

    h = jnp.zeros((C, H), jnp.float32)
    cst = None
    iu0 = og0 = None
    for t in range(CHAIN_LEN - 1, -1, -1):
        waits[t].wait()
        if t == CHAIN_LEN - 1:
            G = jnp.dot(cat_ref[buf, t, :, H:], UW[H:, :4 * H],
                        preferred_element_type=jnp.float32)
            iou_cp.wait()
            c_cp.wait()
            I15 = iou15_scr[buf]
            ig = 0.5 * jnp.tanh(G[:, :H] + 0.5 * I15[:, :H]) + 0.5
            og = 0.5 * jnp.tanh(G[:, H:2 * H] + 0.5 * I15[:, H:2 * H]) + 0.5
            ug = jnp.tanh(G[:, 2 * H:3 * H] + I15[:, 2 * H:3 * H])
            c_prev = c15_scr[buf]
        else:
            cat_ref[buf, t, :, :H] = h
            G = jnp.dot(cat_ref[buf, t], UW[:, :4 * H],
                        preferred_element_type=jnp.float32)
            f_gate = 0.5 * jnp.tanh(G[:, 3 * H:4 * H]) + 0.5
            c_prev = f_gate * cst
            ig, og, ug = gates(G[:, :3 * H])
        iu = ig * ug
        cst = iu + c_prev
        h = og * jnp.tanh(cst)
        if t == 0:
            iu0, og0 = iu, og

    cst = iu0 + cst
    h = og0 * jnp.tanh(cst)

    SID = sid_ref[...]
    SIU = siu_ref[...]
    VC = vc_ref[...]

    def bin_parts(t, hsc):
        mask = (VC[:, t:t + 1] == BC).astype(jnp.float32)
        down = hsc * SID[:, t:t + 1]
        up = hsc * SIU[:, t:t + 1]
        return (jnp.sum(mask, axis=0, keepdims=True),
                jnp.sum(mask * down, axis=0, keepdims=True),
                jnp.sum(mask * up, axis=0, keepdims=True))

    cnt_p = jnp.zeros((1, N_CANDS), jnp.float32)
    pd_p = jnp.zeros((1, N_CANDS), jnp.float32)
    pu_p = jnp.zeros((1, N_CANDS), jnp.float32)
    for t in range(1, CHAIN_LEN):
        cat_ref[buf, t, :, :H] = h
        G = jnp.dot(cat_ref[buf, t], UW, preferred_element_type=jnp.float32)
        f_gate = 0.5 * jnp.tanh(G[:, 3 * H:4 * H]) + 0.5
        c_red = f_gate * cst
        ig, og, ug = gates(G[:, :3 * H])
        cst = ig * ug + c_red
        h = og * jnp.tanh(cst)
        c_t, d_t, u_t = bin_parts(t - 1, G[:, 4 * H:4 * H + 1])
        cnt_p, pd_p, pu_p = cnt_p + c_t, pd_p + d_t, pu_p + u_t
    wl_row = uw_ref[0:H, 4 * H:4 * H + 1]
    blv = uw_ref[2 * H:2 * H + 1, 4 * H:4 * H + 1]
    hsc_last = jnp.dot(h, wl_row, preferred_element_type=jnp.float32) + blv
    c_t, d_t, u_t = bin_parts(CHAIN_LEN - 1, hsc_last)
    cnt_p, pd_p, pu_p = cnt_p + c_t, pd_p + d_t, pu_p + u_t

    acc_ref[0:1, :] = acc_ref[0:1, :] + cnt_p
    acc_ref[1:2, :] = acc_ref[1:2, :] + pd_p
    acc_ref[2:3, :] = acc_ref[2:3, :] + pu_p

    @pl.when(i == nblocks - 1)
    def _finalize():
        cnt = acc_ref[0:1, :]
        denom = jnp.where(cnt == 0.0, 1.0, cnt)
        pd = acc_ref[1:2, :] / denom
        pu = acc_ref[2:3, :] / denom
        score = (1.0 - MU) * pd + MU * jnp.maximum(pd, pu)
        sc = jnp.where(cnt == 0.0, 0.0, score)
        scores_ref[...] = sc
        mx = jnp.max(sc)
        idxs = jax.lax.broadcasted_iota(jnp.int32, (1, N_CANDS), 1)
        best_idx = jnp.min(jnp.where(sc == mx, idxs, jnp.int32(1 << 30)))
        bv_ref[0, 0] = jnp.sum(jnp.where(idxs == best_idx, BC, 0))


def kernel(feature, h, c, iou, scaled_improvement_down, scaled_improvement_up,
           variable_chosen, branch_cands, W_iou, U_iou, b_iou, W_f, W_f_bias,
           b_f, U_f, U_f_bias, W_lin, b_lin):
    n = feature.shape[0]
    nchains = n // CHAIN_LEN
    C = min(512, nchains)
    G = nchains // C

    f3 = feature.reshape(nchains, CHAIN_LEN, H)
    iou3 = iou.reshape(nchains, CHAIN_LEN, 3 * H)
    c3 = c.reshape(nchains, CHAIN_LEN, H)
    sid = scaled_improvement_down.reshape(nchains, CHAIN_LEN)
    siu = scaled_improvement_up.reshape(nchains, CHAIN_LEN)
    vc = variable_chosen.reshape(nchains, CHAIN_LEN)
    bc2 = branch_cands.reshape(1, N_CANDS)

    wfb2 = W_f_bias.reshape(1, H)
    ufb2 = U_f_bias.reshape(1, H)
    wlt = W_lin.reshape(H, 1)
    blin2 = b_lin.reshape(1, 1)

    scores2, bv = pl.pallas_call(
        _body,
        grid=(G,),
        in_specs=[
            pl.BlockSpec(memory_space=pltpu.MemorySpace.HBM),
            pl.BlockSpec(memory_space=pltpu.MemorySpace.HBM),
            pl.BlockSpec(memory_space=pltpu.MemorySpace.HBM),
            pl.BlockSpec((C, CHAIN_LEN), lambda i: (i, 0)),
            pl.BlockSpec((C, CHAIN_LEN), lambda i: (i, 0)),
            pl.BlockSpec((C, CHAIN_LEN), lambda i: (i, 0)),
            pl.BlockSpec((1, N_CANDS), lambda i: (0, 0)),
            pl.BlockSpec((3 * H, H), lambda i: (0, 0)),
            pl.BlockSpec((3 * H, H), lambda i: (0, 0)),
            pl.BlockSpec((1, 3 * H), lambda i: (0, 0)),
            pl.BlockSpec((H, H), lambda i: (0, 0)),
            pl.BlockSpec((1, H), lambda i: (0, 0)),
            pl.BlockSpec((1, H), lambda i: (0, 0)),
            pl.BlockSpec((H, H), lambda i: (0, 0)),
            pl.BlockSpec((1, H), lambda i: (0, 0)),
            pl.BlockSpec((H, 1), lambda i: (0, 0)),
            pl.BlockSpec(memory_space=pltpu.SMEM),
        ],
        out_specs=[
            pl.BlockSpec((1, N_CANDS), lambda i: (0, 0)),
            pl.BlockSpec(memory_space=pltpu.SMEM),
        ],
        out_shape=[
            jax.ShapeDtypeStruct((1, N_CANDS), jnp.float32),
            jax.ShapeDtypeStruct((1, 1), jnp.int32),
        ],
        scratch_shapes=[
            pltpu.VMEM((2, CHAIN_LEN, C, K), jnp.float32),
            pltpu.VMEM((K, NOUT), jnp.float32),
            pltpu.VMEM((2, C, 3 * H), jnp.float32),
            pltpu.VMEM((2, C, H), jnp.float32),
            pltpu.VMEM((8, N_CANDS), jnp.float32),
            pltpu.SemaphoreType.DMA((2, CHAIN_LEN + 2)),
        ],
        compiler_params=pltpu.CompilerParams(
            dimension_semantics=("arbitrary",),
            vmem_limit_bytes=63 * 1024 * 1024),
    )(f3, iou3, c3, sid, siu, vc, bc2, W_iou, U_iou, b_iou,
      W_f, wfb2, b_f, U_f, ufb2, wlt, blin2)

    return bv[0, 0], scores2[0]


# speedup vs baseline: 1.1090x; 1.1090x over previous
import jax
import jax.numpy as jnp
from jax.experimental import pallas as pl
from jax.experimental.pallas import tpu as pltpu

CHAIN_LEN = 16
H = 128
N_CANDS = 64
MU = 0.5
K = 3 * H
NOUT = 5 * H


def _body(f_hbm, iou_hbm, c_hbm, sid_ref, siu_ref, vc_ref, bc_ref,
          wiou_ref, uiou_ref, biou_ref, wf_ref, wfb_ref, bf_ref, uf_ref,
          ufb_ref, wlt_ref, blin_ref,
          scores_ref, bv_ref, cat_ref, uw_ref, iou15_scr, c15_scr, s15_scr,
          acc_ref, sems, sems2):
    i = pl.program_id(0)
    nblocks = pl.num_programs(0)
    C = iou15_scr.shape[1]
    BC = bc_ref[...]
    buf = jax.lax.rem(i, 2)
    nxt = jax.lax.rem(i + 1, 2)

    def head_copies(blk, b):
        return (
            pltpu.make_async_copy(
                iou_hbm.at[pl.ds(blk * C, C), CHAIN_LEN - 1, :],
                iou15_scr.at[b], sems2.at[b, 0]),
            pltpu.make_async_copy(
                c_hbm.at[pl.ds(blk * C, C), CHAIN_LEN - 1, :],
                c15_scr.at[b], sems2.at[b, 1]),
            pltpu.make_async_copy(
                f_hbm.at[pl.ds(blk * C, C), CHAIN_LEN - 1, :],
                s15_scr.at[b, :, 0:H], sems2.at[b, 2]),
        )

    @pl.when(i == 0)
    def _stage_first():
        for cp in head_copies(0, 0):
            cp.start()

    @pl.when(i + 1 < nblocks)
    def _prefetch_next():
        for cp in head_copies(i + 1, nxt):
            cp.start()

    @pl.when(i == 0)
    def _init_const():
        lane = jax.lax.broadcasted_iota(jnp.int32, (C, H), 1)
        onehot = jnp.where(lane == 0, 1.0, 0.0)
        for t in range(CHAIN_LEN):
            cat_ref[t, :, 2 * H:3 * H] = onehot
        s15_scr[0, :, H:2 * H] = onehot
        s15_scr[1, :, H:2 * H] = onehot
        acc_ref[...] = jnp.zeros_like(acc_ref)

        lane3 = jax.lax.broadcasted_iota(jnp.int32, (H, 3 * H), 1)
        sub3 = jax.lax.broadcasted_iota(jnp.int32, (H, 3 * H), 0)
        lane1 = jax.lax.broadcasted_iota(jnp.int32, (H, H), 1)
        sub1 = jax.lax.broadcasted_iota(jnp.int32, (H, H), 0)
        sc3 = jnp.where(lane3 < 2 * H, 0.5, 1.0)
        uw_ref[0:H, 0:3 * H] = jnp.swapaxes(uiou_ref[...], 0, 1) * sc3
        uw_ref[0:H, 3 * H:4 * H] = jnp.swapaxes(uf_ref[...], 0, 1) * 0.5
        uw_ref[0:H, 4 * H:5 * H] = jnp.where(
            lane1 == 0, jnp.broadcast_to(wlt_ref[...], (H, H)), 0.0)
        uw_ref[H:2 * H, 0:3 * H] = jnp.swapaxes(wiou_ref[...], 0, 1) * sc3
        uw_ref[H:2 * H, 3 * H:4 * H] = jnp.swapaxes(wf_ref[...], 0, 1) * 0.5
        uw_ref[H:2 * H, 4 * H:5 * H] = jnp.zeros((H, H), jnp.float32)
        biou_row = biou_ref[...] * jnp.where(
            lane3[0:1, :] < 2 * H, 0.5, 1.0)
        uw_ref[2 * H:3 * H, 0:3 * H] = jnp.where(
            sub3 == 0, jnp.broadcast_to(biou_row, (H, 3 * H)), 0.0)
        bias_f = (wfb_ref[...] + ufb_ref[...] + bf_ref[...]) * 0.5
        uw_ref[2 * H:3 * H, 3 * H:4 * H] = jnp.where(
            sub1 == 0, jnp.broadcast_to(bias_f, (H, H)), 0.0)
        uw_ref[2 * H:3 * H, 4 * H:5 * H] = jnp.where(
            (sub1 == 0) & (lane1 == 0), blin_ref[0, 0], 0.0)

    UW = uw_ref[...]

    iou_cp, c_cp, f15_cp = head_copies(i, buf)

    copies = []
    for t in range(CHAIN_LEN - 1, -1, -1):
        cp = pltpu.make_async_copy(
            f_hbm.at[pl.ds(i * C, C), t, :],
            cat_ref.at[t, :, H:2 * H],
            sems.at[t])
        cp.start()
        copies.append((t, cp))
    waits = dict(copies)

    def gates(g3):
        ig = 0.5 * jnp.tanh(g3[:, :H]) + 0.5
        og = 0.5 * jnp.tanh(g3[:, H:2 * H]) + 0.5
        ug = jnp.tanh(g3[:, 2 * H:3 * H])
        return ig, og, ug

    h = jnp.zeros((C, H), jnp.float32)
    cst = None
    iu0 = og0 = None
    for t in range(CHAIN_LEN - 1, -1, -1):
        if t == CHAIN_LEN - 1:
            f15_cp.wait()
            G = jnp.dot(s15_scr[buf], UW[H:, :4 * H],
                        preferred_element_type=jnp.float32)
            iou_cp.wait()
            c_cp.wait()
            I15 = iou15_scr[buf]
            ig = 0.5 * jnp.tanh(G[:, :H] + 0.5 * I15[:, :H]) + 0.5
            og = 0.5 * jnp.tanh(G[:, H:2 * H] + 0.5 * I15[:, H:2 * H]) + 0.5
            ug = jnp.tanh(G[:, 2 * H:3 * H] + I15[:, 2 * H:3 * H])
            c_prev = c15_scr[buf]
        else:
            waits[t].wait()
            cat_ref[t, :, :H] = h
            G = jnp.dot(cat_ref[t], UW[:, :4 * H],
                        preferred_element_type=jnp.float32)
            f_gate = 0.5 * jnp.tanh(G[:, 3 * H:4 * H]) + 0.5
            c_prev = f_gate * cst
            ig, og, ug = gates(G[:, :3 * H])
        iu = ig * ug
        cst = iu + c_prev
        h = og * jnp.tanh(cst)
        if t == 0:
            iu0, og0 = iu, og

    cst = iu0 + cst
    h = og0 * jnp.tanh(cst)

    SID = sid_ref[...]
    SIU = siu_ref[...]
    VC = vc_ref[...]

    def bin_parts(t, hsc):
        mask = (VC[:, t:t + 1] == BC).astype(jnp.float32)
        down = hsc * SID[:, t:t + 1]
        up = hsc * SIU[:, t:t + 1]
        return (jnp.sum(mask, axis=0, keepdims=True),
                jnp.sum(mask * down, axis=0, keepdims=True),
                jnp.sum(mask * up, axis=0, keepdims=True))

    cnt_p = jnp.zeros((1, N_CANDS), jnp.float32)
    pd_p = jnp.zeros((1, N_CANDS), jnp.float32)
    pu_p = jnp.zeros((1, N_CANDS), jnp.float32)
    for t in range(1, CHAIN_LEN):
        if t == CHAIN_LEN - 1:
            waits[t].wait()
        cat_ref[t, :, :H] = h
        G = jnp.dot(cat_ref[t], UW, preferred_element_type=jnp.float32)
        f_gate = 0.5 * jnp.tanh(G[:, 3 * H:4 * H]) + 0.5
        c_red = f_gate * cst
        ig, og, ug = gates(G[:, :3 * H])
        cst = ig * ug + c_red
        h = og * jnp.tanh(cst)
        c_t, d_t, u_t = bin_parts(t - 1, G[:, 4 * H:4 * H + 1])
        cnt_p, pd_p, pu_p = cnt_p + c_t, pd_p + d_t, pu_p + u_t
    wl_row = uw_ref[0:H, 4 * H:4 * H + 1]
    blv = uw_ref[2 * H:2 * H + 1, 4 * H:4 * H + 1]
    hsc_last = jnp.dot(h, wl_row, preferred_element_type=jnp.float32) + blv
    c_t, d_t, u_t = bin_parts(CHAIN_LEN - 1, hsc_last)
    cnt_p, pd_p, pu_p = cnt_p + c_t, pd_p + d_t, pu_p + u_t

    acc_ref[0:1, :] = acc_ref[0:1, :] + cnt_p
    acc_ref[1:2, :] = acc_ref[1:2, :] + pd_p
    acc_ref[2:3, :] = acc_ref[2:3, :] + pu_p

    @pl.when(i == nblocks - 1)
    def _finalize():
        cnt = acc_ref[0:1, :]
        denom = jnp.where(cnt == 0.0, 1.0, cnt)
        pd = acc_ref[1:2, :] / denom
        pu = acc_ref[2:3, :] / denom
        score = (1.0 - MU) * pd + MU * jnp.maximum(pd, pu)
        sc = jnp.where(cnt == 0.0, 0.0, score)
        scores_ref[...] = sc
        mx = jnp.max(sc)
        idxs = jax.lax.broadcasted_iota(jnp.int32, (1, N_CANDS), 1)
        best_idx = jnp.min(jnp.where(sc == mx, idxs, jnp.int32(1 << 30)))
        bv_ref[0, 0] = jnp.sum(jnp.where(idxs == best_idx, BC, 0))


def kernel(feature, h, c, iou, scaled_improvement_down, scaled_improvement_up,
           variable_chosen, branch_cands, W_iou, U_iou, b_iou, W_f, W_f_bias,
           b_f, U_f, U_f_bias, W_lin, b_lin):
    n = feature.shape[0]
    nchains = n // CHAIN_LEN
    C = min(1024, nchains)
    G = nchains // C

    f3 = feature.reshape(nchains, CHAIN_LEN, H)
    iou3 = iou.reshape(nchains, CHAIN_LEN, 3 * H)
    c3 = c.reshape(nchains, CHAIN_LEN, H)
    sid = scaled_improvement_down.reshape(nchains, CHAIN_LEN)
    siu = scaled_improvement_up.reshape(nchains, CHAIN_LEN)
    vc = variable_chosen.reshape(nchains, CHAIN_LEN)
    bc2 = branch_cands.reshape(1, N_CANDS)

    wfb2 = W_f_bias.reshape(1, H)
    ufb2 = U_f_bias.reshape(1, H)
    wlt = W_lin.reshape(H, 1)
    blin2 = b_lin.reshape(1, 1)

    scores2, bv = pl.pallas_call(
        _body,
        grid=(G,),
        in_specs=[
            pl.BlockSpec(memory_space=pltpu.MemorySpace.HBM),
            pl.BlockSpec(memory_space=pltpu.MemorySpace.HBM),
            pl.BlockSpec(memory_space=pltpu.MemorySpace.HBM),
            pl.BlockSpec((C, CHAIN_LEN), lambda i: (i, 0)),
            pl.BlockSpec((C, CHAIN_LEN), lambda i: (i, 0)),
            pl.BlockSpec((C, CHAIN_LEN), lambda i: (i, 0)),
            pl.BlockSpec((1, N_CANDS), lambda i: (0, 0)),
            pl.BlockSpec((3 * H, H), lambda i: (0, 0)),
            pl.BlockSpec((3 * H, H), lambda i: (0, 0)),
            pl.BlockSpec((1, 3 * H), lambda i: (0, 0)),
            pl.BlockSpec((H, H), lambda i: (0, 0)),
            pl.BlockSpec((1, H), lambda i: (0, 0)),
            pl.BlockSpec((1, H), lambda i: (0, 0)),
            pl.BlockSpec((H, H), lambda i: (0, 0)),
            pl.BlockSpec((1, H), lambda i: (0, 0)),
            pl.BlockSpec((H, 1), lambda i: (0, 0)),
            pl.BlockSpec(memory_space=pltpu.SMEM),
        ],
        out_specs=[
            pl.BlockSpec((1, N_CANDS), lambda i: (0, 0)),
            pl.BlockSpec(memory_space=pltpu.SMEM),
        ],
        out_shape=[
            jax.ShapeDtypeStruct((1, N_CANDS), jnp.float32),
            jax.ShapeDtypeStruct((1, 1), jnp.int32),
        ],
        scratch_shapes=[
            pltpu.VMEM((CHAIN_LEN, C, K), jnp.float32),
            pltpu.VMEM((K, NOUT), jnp.float32),
            pltpu.VMEM((2, C, 3 * H), jnp.float32),
            pltpu.VMEM((2, C, H), jnp.float32),
            pltpu.VMEM((2, C, 2 * H), jnp.float32),
            pltpu.VMEM((8, N_CANDS), jnp.float32),
            pltpu.SemaphoreType.DMA((CHAIN_LEN,)),
            pltpu.SemaphoreType.DMA((2, 3)),
        ],
        compiler_params=pltpu.CompilerParams(
            dimension_semantics=("arbitrary",),
            vmem_limit_bytes=63 * 1024 * 1024),
    )(f3, iou3, c3, sid, siu, vc, bc2, W_iou, U_iou, b_iou,
      W_f, wfb2, b_f, U_f, ufb2, wlt, blin2)

    return bv[0, 0], scores2[0]
